# Initial kernel scaffold; baseline (speedup 1.0000x reference)
import jax
import jax.numpy as jnp
from jax.experimental import pallas as pl


def kernel(x, edge_index, Wl1, bl1, Wr1, Wl2, bl2, Wr2, Wl3, bl3, Wr3, Wc, bc):
    raise NotImplementedError("write your pallas kernel here")


# speedup vs baseline: 4.7088x; 4.7088x over previous
import functools

import jax
import jax.numpy as jnp
from jax import lax
from jax.experimental import pallas as pl
from jax.experimental.pallas import tpu as pltpu
from jax.experimental.pallas import tpu_sc as plsc

_NC = 2
_NS = 16
_NW = _NC * _NS
_CHUNK = 80
_ZR = 48
_BLK = 1024


def _row_partition(n_nodes):
    rpt = (n_nodes // _NS) // _ZR * _ZR
    tail = n_nodes - rpt * _NS
    assert rpt % _ZR == 0 and tail % 8 == 0 and tail <= _ZR
    return rpt, tail


@functools.cache
def _make_agg(n_nodes, n_edges, d):
    epw = n_edges // _NW
    n_chunks = epw // _CHUNK
    rpt, tail = _row_partition(n_nodes)
    assert epw * _NW == n_edges and n_chunks * _CHUNK == epw

    mesh = plsc.VectorSubcoreMesh(core_axis_name="c", subcore_axis_name="s")
    out_type = jax.ShapeDtypeStruct((_NC, n_nodes, d), jnp.float32)
    scratch = [
        pltpu.VMEM_SHARED((n_nodes, d), jnp.float32),
        pltpu.VMEM((_CHUNK,), jnp.int32),
        pltpu.VMEM((_CHUNK,), jnp.int32),
        pltpu.VMEM((_CHUNK, d), jnp.float32),
        pltpu.VMEM((_ZR, d), jnp.float32),
        pltpu.SemaphoreType.DMA,
    ]

    def body(src, dst, z, out, acc, sidx, didx, rows, zbuf, sem):
        cid = lax.axis_index("c")
        sid = lax.axis_index("s")
        wid = sid * _NC + cid
        zero16 = jnp.zeros((16,), jnp.float32)
        vregs_per_row = d // 16

        def zfill(i, carry):
            zbuf[i // vregs_per_row, pl.ds((i % vregs_per_row) * 16, 16)] = zero16
            return carry

        lax.fori_loop(0, _ZR * vregs_per_row, zfill, 0)

        row0 = sid * rpt
        is_last = sid == _NS - 1
        for r in range(rpt // _ZR):
            pltpu.sync_copy(zbuf, acc.at[pl.ds(row0 + r * _ZR, _ZR)])
        if tail:
            @pl.when(is_last)
            def _():
                pltpu.sync_copy(zbuf.at[pl.ds(0, tail)],
                                acc.at[pl.ds(rpt * _NS, tail)])

        plsc.subcore_barrier()

        base = wid * epw

        def chunk(j, carry):
            off = base + j * _CHUNK
            pltpu.sync_copy(src.at[pl.ds(off, _CHUNK)], sidx)
            pltpu.sync_copy(dst.at[pl.ds(off, _CHUNK)], didx)
            pltpu.async_copy(z.at[sidx], rows, sem).wait()
            pltpu.sync_copy(rows, acc.at[didx], add=True)
            return carry

        lax.fori_loop(0, n_chunks, chunk, 0)

        plsc.subcore_barrier()
        pltpu.sync_copy(acc.at[pl.ds(row0, rpt)], out.at[cid, pl.ds(row0, rpt)])
        if tail:
            @pl.when(is_last)
            def _():
                pltpu.sync_copy(acc.at[pl.ds(rpt * _NS, tail)],
                                out.at[cid, pl.ds(rpt * _NS, tail)])

    return pl.kernel(body, out_type=out_type, mesh=mesh,
                     scratch_types=scratch)


@functools.cache
def _make_cnt(n_nodes, n_edges, d):
    epw = n_edges // _NW
    n_chunks = epw // _CHUNK
    rpt, tail = _row_partition(n_nodes)
    assert epw * _NW == n_edges and n_chunks * _CHUNK == epw

    mesh = plsc.VectorSubcoreMesh(core_axis_name="c", subcore_axis_name="s")
    out_type = jax.ShapeDtypeStruct((_NC, n_nodes, d), jnp.float32)
    scratch = [
        pltpu.VMEM_SHARED((n_nodes, d), jnp.float32),
        pltpu.VMEM((_CHUNK,), jnp.int32),
        pltpu.VMEM((_CHUNK, d), jnp.float32),
        pltpu.VMEM((_ZR, d), jnp.float32),
    ]

    def body(dst, out, acc, didx, ones_rows, zbuf):
        cid = lax.axis_index("c")
        sid = lax.axis_index("s")
        wid = sid * _NC + cid
        zero16 = jnp.zeros((16,), jnp.float32)
        one16 = jnp.ones((16,), jnp.float32)
        vregs_per_row = d // 16

        def zfill(i, carry):
            zbuf[i // vregs_per_row, pl.ds((i % vregs_per_row) * 16, 16)] = zero16
            return carry

        lax.fori_loop(0, _ZR * vregs_per_row, zfill, 0)

        def ofill(i, carry):
            ones_rows[i // vregs_per_row,
                      pl.ds((i % vregs_per_row) * 16, 16)] = one16
            return carry

        lax.fori_loop(0, _CHUNK * vregs_per_row, ofill, 0)

        row0 = sid * rpt
        is_last = sid == _NS - 1
        for r in range(rpt // _ZR):
            pltpu.sync_copy(zbuf, acc.at[pl.ds(row0 + r * _ZR, _ZR)])
        if tail:
            @pl.when(is_last)
            def _():
                pltpu.sync_copy(zbuf.at[pl.ds(0, tail)],
                                acc.at[pl.ds(rpt * _NS, tail)])

        plsc.subcore_barrier()

        base = wid * epw

        def chunk(j, carry):
            off = base + j * _CHUNK
            pltpu.sync_copy(dst.at[pl.ds(off, _CHUNK)], didx)
            pltpu.sync_copy(ones_rows, acc.at[didx], add=True)
            return carry

        lax.fori_loop(0, n_chunks, chunk, 0)

        plsc.subcore_barrier()
        pltpu.sync_copy(acc.at[pl.ds(row0, rpt)], out.at[cid, pl.ds(row0, rpt)])
        if tail:
            @pl.when(is_last)
            def _():
                pltpu.sync_copy(acc.at[pl.ds(rpt * _NS, tail)],
                                out.at[cid, pl.ds(rpt * _NS, tail)])

    return pl.kernel(body, out_type=out_type, mesh=mesh,
                     scratch_types=scratch)


def _pre_body(x_ref, wl_ref, wr_ref, z_ref, s_ref):
    xb = x_ref[...]
    z_ref[...] = jnp.dot(xb, wl_ref[...], preferred_element_type=jnp.float32)
    s_ref[...] = jnp.dot(xb, wr_ref[...], preferred_element_type=jnp.float32)


@functools.cache
def _make_pre(n, d):
    grid = pl.cdiv(n, _BLK)
    return pl.pallas_call(
        _pre_body,
        grid=(grid,),
        in_specs=[pl.BlockSpec((_BLK, d), lambda i: (i, 0)),
                  pl.BlockSpec((d, d), lambda i: (0, 0)),
                  pl.BlockSpec((d, d), lambda i: (0, 0))],
        out_specs=[pl.BlockSpec((_BLK, d), lambda i: (i, 0)),
                   pl.BlockSpec((_BLK, d), lambda i: (i, 0))],
        out_shape=[jax.ShapeDtypeStruct((n, d), jnp.float32),
                   jax.ShapeDtypeStruct((n, d), jnp.float32)],
    )


def _mid_body(parts_ref, cnts_ref, sprev_ref, b_ref, wl_ref, wr_ref,
              h_ref, z_ref, s_ref):
    inv = 1.0 / jnp.maximum(cnts_ref[0] + cnts_ref[1], 1.0)
    mean = (parts_ref[0] + parts_ref[1]) * inv
    h = jnp.maximum(mean + b_ref[...] + sprev_ref[...], 0.0)
    h_ref[...] = h
    z_ref[...] = jnp.dot(h, wl_ref[...], preferred_element_type=jnp.float32)
    s_ref[...] = jnp.dot(h, wr_ref[...], preferred_element_type=jnp.float32)


@functools.cache
def _make_mid(n, d):
    grid = pl.cdiv(n, _BLK)
    return pl.pallas_call(
        _mid_body,
        grid=(grid,),
        in_specs=[pl.BlockSpec((_NC, _BLK, d), lambda i: (0, i, 0)),
                  pl.BlockSpec((_NC, _BLK, d), lambda i: (0, i, 0)),
                  pl.BlockSpec((_BLK, d), lambda i: (i, 0)),
                  pl.BlockSpec((1, d), lambda i: (0, 0)),
                  pl.BlockSpec((d, d), lambda i: (0, 0)),
                  pl.BlockSpec((d, d), lambda i: (0, 0))],
        out_specs=[pl.BlockSpec((_BLK, d), lambda i: (i, 0)),
                   pl.BlockSpec((_BLK, d), lambda i: (i, 0)),
                   pl.BlockSpec((_BLK, d), lambda i: (i, 0))],
        out_shape=[jax.ShapeDtypeStruct((n, d), jnp.float32),
                   jax.ShapeDtypeStruct((n, d), jnp.float32),
                   jax.ShapeDtypeStruct((n, d), jnp.float32)],
    )


def _fin_body(parts_ref, cnts_ref, s3_ref, b_ref, h2_ref, wc_ref, bc_ref,
              o_ref):
    inv = 1.0 / jnp.maximum(cnts_ref[0] + cnts_ref[1], 1.0)
    mean = (parts_ref[0] + parts_ref[1]) * inv
    h3 = jnp.maximum(mean + b_ref[...] + s3_ref[...], 0.0)
    v = h3 + h2_ref[...]
    o_ref[...] = (jnp.dot(v, wc_ref[...], preferred_element_type=jnp.float32)
                  + bc_ref[...])


@functools.cache
def _make_fin(n, d, ncls):
    grid = pl.cdiv(n, _BLK)
    return pl.pallas_call(
        _fin_body,
        grid=(grid,),
        in_specs=[pl.BlockSpec((_NC, _BLK, d), lambda i: (0, i, 0)),
                  pl.BlockSpec((_NC, _BLK, d), lambda i: (0, i, 0)),
                  pl.BlockSpec((_BLK, d), lambda i: (i, 0)),
                  pl.BlockSpec((1, d), lambda i: (0, 0)),
                  pl.BlockSpec((_BLK, d), lambda i: (i, 0)),
                  pl.BlockSpec((d, ncls), lambda i: (0, 0)),
                  pl.BlockSpec((1, ncls), lambda i: (0, 0))],
        out_specs=pl.BlockSpec((_BLK, ncls), lambda i: (i, 0)),
        out_shape=jax.ShapeDtypeStruct((n, ncls), jnp.float32),
    )


def kernel(x, edge_index, Wl1, bl1, Wr1, Wl2, bl2, Wr2, Wl3, bl3, Wr3,
           Wc, bc):
    n, d = x.shape
    e = edge_index.shape[1]
    ncls = Wc.shape[1]
    src = edge_index[0]
    dst = edge_index[1]

    pre = _make_pre(n, d)
    agg = _make_agg(n, e, d)
    cnt = _make_cnt(n, e, d)
    mid = _make_mid(n, d)
    fin = _make_fin(n, d, ncls)

    cnts = cnt(dst)
    z1, s1 = pre(x, Wl1, Wr1)
    p1 = agg(src, dst, z1)
    _h1, z2, s2 = mid(p1, cnts, s1, bl1.reshape(1, d), Wl2, Wr2)
    p2 = agg(src, dst, z2)
    h2, z3, s3 = mid(p2, cnts, s2, bl2.reshape(1, d), Wl3, Wr3)
    p3 = agg(src, dst, z3)
    return fin(p3, cnts, s3, bl3.reshape(1, d), h2, Wc, bc.reshape(1, ncls))


# speedup vs baseline: 7.0477x; 1.4967x over previous
import functools

import jax
import jax.numpy as jnp
from jax import lax
from jax.experimental import pallas as pl
from jax.experimental.pallas import tpu as pltpu
from jax.experimental.pallas import tpu_sc as plsc

_NC = 2
_NS = 16
_NW = _NC * _NS
_CHUNK = 80
_ZR = 48
_BLK = 1024


def _row_partition(n_nodes):
    rpt = (n_nodes // _NS) // _ZR * _ZR
    tail = n_nodes - rpt * _NS
    assert rpt % _ZR == 0 and tail % 8 == 0 and tail <= _ZR
    return rpt, tail


@functools.cache
def _make_agg(n_nodes, n_edges, d):
    epw = n_edges // _NW
    n_chunks = epw // _CHUNK
    rpt, tail = _row_partition(n_nodes)
    assert epw * _NW == n_edges and n_chunks * _CHUNK == epw

    assert n_chunks >= 3 and n_chunks % 2 == 1

    mesh = plsc.VectorSubcoreMesh(core_axis_name="c", subcore_axis_name="s")
    out_type = jax.ShapeDtypeStruct((_NC, n_nodes, d), jnp.float32)
    scratch = [
        pltpu.VMEM_SHARED((n_nodes, d), jnp.float32),
        pltpu.VMEM((_CHUNK,), jnp.int32),
        pltpu.VMEM((_CHUNK,), jnp.int32),
        pltpu.VMEM((_CHUNK, d), jnp.float32),
        pltpu.VMEM((_CHUNK,), jnp.int32),
        pltpu.VMEM((_CHUNK,), jnp.int32),
        pltpu.VMEM((_CHUNK, d), jnp.float32),
        pltpu.VMEM((_ZR, d), jnp.float32),
        pltpu.SemaphoreType.DMA,
        pltpu.SemaphoreType.DMA,
    ]

    def body(src, dst, z, out, acc, sidx, didx, rows,
             sidx_b, didx_b, rows_b, zbuf, sem, sem_b):
        cid = lax.axis_index("c")
        sid = lax.axis_index("s")
        wid = sid * _NC + cid
        zero16 = jnp.zeros((16,), jnp.float32)
        vregs_per_row = d // 16

        def zfill(i, carry):
            zbuf[i // vregs_per_row, pl.ds((i % vregs_per_row) * 16, 16)] = zero16
            return carry

        lax.fori_loop(0, _ZR * vregs_per_row, zfill, 0)

        row0 = sid * rpt
        is_last = sid == _NS - 1
        for r in range(rpt // _ZR):
            pltpu.sync_copy(zbuf, acc.at[pl.ds(row0 + r * _ZR, _ZR)])
        if tail:
            @pl.when(is_last)
            def _():
                pltpu.sync_copy(zbuf.at[pl.ds(0, tail)],
                                acc.at[pl.ds(rpt * _NS, tail)])

        plsc.subcore_barrier()

        base = wid * epw

        def load_idx(c, si, di):
            off = base + c * _CHUNK
            pltpu.sync_copy(src.at[pl.ds(off, _CHUNK)], si)
            pltpu.sync_copy(dst.at[pl.ds(off, _CHUNK)], di)

        load_idx(0, sidx, didx)
        pltpu.async_copy(z.at[sidx], rows, sem)

        def pipe(m, carry):
            c0 = 2 * m
            load_idx(c0 + 1, sidx_b, didx_b)
            gat_b = pltpu.async_copy(z.at[sidx_b], rows_b, sem_b)
            pltpu.make_async_copy(z.at[sidx], rows, sem).wait()
            pltpu.sync_copy(rows, acc.at[didx], add=True)
            load_idx(c0 + 2, sidx, didx)
            pltpu.async_copy(z.at[sidx], rows, sem)
            gat_b.wait()
            pltpu.sync_copy(rows_b, acc.at[didx_b], add=True)
            return carry

        lax.fori_loop(0, (n_chunks - 1) // 2, pipe, 0)
        pltpu.make_async_copy(z.at[sidx], rows, sem).wait()
        pltpu.sync_copy(rows, acc.at[didx], add=True)

        plsc.subcore_barrier()
        pltpu.sync_copy(acc.at[pl.ds(row0, rpt)], out.at[cid, pl.ds(row0, rpt)])
        if tail:
            @pl.when(is_last)
            def _():
                pltpu.sync_copy(acc.at[pl.ds(rpt * _NS, tail)],
                                out.at[cid, pl.ds(rpt * _NS, tail)])

    return pl.kernel(body, out_type=out_type, mesh=mesh,
                     scratch_types=scratch)


@functools.cache
def _make_cnt(n_nodes, n_edges, d):
    epw = n_edges // _NW
    n_chunks = epw // _CHUNK
    rpt, tail = _row_partition(n_nodes)
    assert epw * _NW == n_edges and n_chunks * _CHUNK == epw

    mesh = plsc.VectorSubcoreMesh(core_axis_name="c", subcore_axis_name="s")
    out_type = jax.ShapeDtypeStruct((_NC, n_nodes, d), jnp.float32)
    scratch = [
        pltpu.VMEM_SHARED((n_nodes, d), jnp.float32),
        pltpu.VMEM((_CHUNK,), jnp.int32),
        pltpu.VMEM((_CHUNK, d), jnp.float32),
        pltpu.VMEM((_ZR, d), jnp.float32),
    ]

    def body(dst, out, acc, didx, ones_rows, zbuf):
        cid = lax.axis_index("c")
        sid = lax.axis_index("s")
        wid = sid * _NC + cid
        zero16 = jnp.zeros((16,), jnp.float32)
        one16 = jnp.ones((16,), jnp.float32)
        vregs_per_row = d // 16

        def zfill(i, carry):
            zbuf[i // vregs_per_row, pl.ds((i % vregs_per_row) * 16, 16)] = zero16
            return carry

        lax.fori_loop(0, _ZR * vregs_per_row, zfill, 0)

        def ofill(i, carry):
            ones_rows[i // vregs_per_row,
                      pl.ds((i % vregs_per_row) * 16, 16)] = one16
            return carry

        lax.fori_loop(0, _CHUNK * vregs_per_row, ofill, 0)

        row0 = sid * rpt
        is_last = sid == _NS - 1
        for r in range(rpt // _ZR):
            pltpu.sync_copy(zbuf, acc.at[pl.ds(row0 + r * _ZR, _ZR)])
        if tail:
            @pl.when(is_last)
            def _():
                pltpu.sync_copy(zbuf.at[pl.ds(0, tail)],
                                acc.at[pl.ds(rpt * _NS, tail)])

        plsc.subcore_barrier()

        base = wid * epw

        def chunk(j, carry):
            off = base + j * _CHUNK
            pltpu.sync_copy(dst.at[pl.ds(off, _CHUNK)], didx)
            pltpu.sync_copy(ones_rows, acc.at[didx], add=True)
            return carry

        lax.fori_loop(0, n_chunks, chunk, 0)

        plsc.subcore_barrier()
        pltpu.sync_copy(acc.at[pl.ds(row0, rpt)], out.at[cid, pl.ds(row0, rpt)])
        if tail:
            @pl.when(is_last)
            def _():
                pltpu.sync_copy(acc.at[pl.ds(rpt * _NS, tail)],
                                out.at[cid, pl.ds(rpt * _NS, tail)])

    return pl.kernel(body, out_type=out_type, mesh=mesh,
                     scratch_types=scratch)


def _pre_body(x_ref, wl_ref, wr_ref, z_ref, s_ref):
    xb = x_ref[...]
    z_ref[...] = jnp.dot(xb, wl_ref[...], preferred_element_type=jnp.float32)
    s_ref[...] = jnp.dot(xb, wr_ref[...], preferred_element_type=jnp.float32)


@functools.cache
def _make_pre(n, d):
    grid = pl.cdiv(n, _BLK)
    return pl.pallas_call(
        _pre_body,
        grid=(grid,),
        in_specs=[pl.BlockSpec((_BLK, d), lambda i: (i, 0)),
                  pl.BlockSpec((d, d), lambda i: (0, 0)),
                  pl.BlockSpec((d, d), lambda i: (0, 0))],
        out_specs=[pl.BlockSpec((_BLK, d), lambda i: (i, 0)),
                   pl.BlockSpec((_BLK, d), lambda i: (i, 0))],
        out_shape=[jax.ShapeDtypeStruct((n, d), jnp.float32),
                   jax.ShapeDtypeStruct((n, d), jnp.float32)],
    )


def _mid_body(parts_ref, cnts_ref, sprev_ref, b_ref, wl_ref, wr_ref,
              h_ref, z_ref, s_ref):
    inv = 1.0 / jnp.maximum(cnts_ref[0] + cnts_ref[1], 1.0)
    mean = (parts_ref[0] + parts_ref[1]) * inv
    h = jnp.maximum(mean + b_ref[...] + sprev_ref[...], 0.0)
    h_ref[...] = h
    z_ref[...] = jnp.dot(h, wl_ref[...], preferred_element_type=jnp.float32)
    s_ref[...] = jnp.dot(h, wr_ref[...], preferred_element_type=jnp.float32)


@functools.cache
def _make_mid(n, d):
    grid = pl.cdiv(n, _BLK)
    return pl.pallas_call(
        _mid_body,
        grid=(grid,),
        in_specs=[pl.BlockSpec((_NC, _BLK, d), lambda i: (0, i, 0)),
                  pl.BlockSpec((_NC, _BLK, d), lambda i: (0, i, 0)),
                  pl.BlockSpec((_BLK, d), lambda i: (i, 0)),
                  pl.BlockSpec((1, d), lambda i: (0, 0)),
                  pl.BlockSpec((d, d), lambda i: (0, 0)),
                  pl.BlockSpec((d, d), lambda i: (0, 0))],
        out_specs=[pl.BlockSpec((_BLK, d), lambda i: (i, 0)),
                   pl.BlockSpec((_BLK, d), lambda i: (i, 0)),
                   pl.BlockSpec((_BLK, d), lambda i: (i, 0))],
        out_shape=[jax.ShapeDtypeStruct((n, d), jnp.float32),
                   jax.ShapeDtypeStruct((n, d), jnp.float32),
                   jax.ShapeDtypeStruct((n, d), jnp.float32)],
    )


def _fin_body(parts_ref, cnts_ref, s3_ref, b_ref, h2_ref, wc_ref, bc_ref,
              o_ref):
    inv = 1.0 / jnp.maximum(cnts_ref[0] + cnts_ref[1], 1.0)
    mean = (parts_ref[0] + parts_ref[1]) * inv
    h3 = jnp.maximum(mean + b_ref[...] + s3_ref[...], 0.0)
    v = h3 + h2_ref[...]
    o_ref[...] = (jnp.dot(v, wc_ref[...], preferred_element_type=jnp.float32)
                  + bc_ref[...])


@functools.cache
def _make_fin(n, d, ncls):
    grid = pl.cdiv(n, _BLK)
    return pl.pallas_call(
        _fin_body,
        grid=(grid,),
        in_specs=[pl.BlockSpec((_NC, _BLK, d), lambda i: (0, i, 0)),
                  pl.BlockSpec((_NC, _BLK, d), lambda i: (0, i, 0)),
                  pl.BlockSpec((_BLK, d), lambda i: (i, 0)),
                  pl.BlockSpec((1, d), lambda i: (0, 0)),
                  pl.BlockSpec((_BLK, d), lambda i: (i, 0)),
                  pl.BlockSpec((d, ncls), lambda i: (0, 0)),
                  pl.BlockSpec((1, ncls), lambda i: (0, 0))],
        out_specs=pl.BlockSpec((_BLK, ncls), lambda i: (i, 0)),
        out_shape=jax.ShapeDtypeStruct((n, ncls), jnp.float32),
    )


def kernel(x, edge_index, Wl1, bl1, Wr1, Wl2, bl2, Wr2, Wl3, bl3, Wr3,
           Wc, bc):
    n, d = x.shape
    e = edge_index.shape[1]
    ncls = Wc.shape[1]
    src = edge_index[0]
    dst = edge_index[1]

    pre = _make_pre(n, d)
    agg = _make_agg(n, e, d)
    cnt = _make_cnt(n, e, d)
    mid = _make_mid(n, d)
    fin = _make_fin(n, d, ncls)

    cnts = cnt(dst)
    z1, s1 = pre(x, Wl1, Wr1)
    p1 = agg(src, dst, z1)
    _h1, z2, s2 = mid(p1, cnts, s1, bl1.reshape(1, d), Wl2, Wr2)
    p2 = agg(src, dst, z2)
    h2, z3, s3 = mid(p2, cnts, s2, bl2.reshape(1, d), Wl3, Wr3)
    p3 = agg(src, dst, z3)
    return fin(p3, cnts, s3, bl3.reshape(1, d), h2, Wc, bc.reshape(1, ncls))
